# baseline (speedup 1.0000x reference)
import numpy as np
import jax
import jax.numpy as jnp
from jax.experimental import pallas as pl
from jax.experimental.pallas import tpu as pltpu
from jax.experimental.pallas import tpu_sc as plsc

_B = 512
_TB = 1024


def kernel(x, table):
    batch, hist = x.shape
    vocab, dim = table.shape
    xt = x.T

    xt_s = (
        xt.reshape(hist, batch // _TB, 2, _TB // 2)
        .transpose(0, 1, 3, 2)
        .reshape(hist, batch)
    )

    mesh = plsc.VectorSubcoreMesh(core_axis_name="c", subcore_axis_name="s")

    @pl.kernel(
        out_type=jax.ShapeDtypeStruct((hist, batch, dim), table.dtype),
        mesh=mesh,
        compiler_params=pltpu.CompilerParams(use_tc_tiling_on_sc=False),
    )
    def gather_kernel(table_hbm, i_hbm, o_hbm):
        def body(i_vmem, o_vmem):
            pltpu.sync_copy(table_hbm.at[i_vmem.at[0]], o_vmem.at[0])

        pltpu.emit_pipeline(
            body,
            grid=(hist, batch // _B),
            in_specs=[pl.BlockSpec((1, _B), index_map=lambda h, b: (h, b))],
            out_specs=[
                pl.BlockSpec((1, _B, dim), index_map=lambda h, b: (h, b, 0))
            ],
            core_axis_name=("c", "s"),
            dimension_semantics=(pltpu.PARALLEL, pltpu.PARALLEL),
        )(i_hbm, o_hbm)

    g = gather_kernel(table, xt_s)

    g128 = g.reshape(hist, batch // 2, 2 * dim)

    def transpose_body(in_ref, out_ref):
        vt = in_ref[0].T
        out_ref[0] = jnp.concatenate([vt[:dim], vt[dim:]], axis=1)

    out_t = pl.pallas_call(
        transpose_body,
        grid=(hist, batch // _TB),
        in_specs=[
            pl.BlockSpec(
                (1, _TB // 2, 2 * dim), index_map=lambda h, b: (h, b, 0)
            )
        ],
        out_specs=pl.BlockSpec((1, dim, _TB), index_map=lambda h, b: (h, 0, b)),
        out_shape=jax.ShapeDtypeStruct((hist, dim, batch), table.dtype),
    )(g128)

    return out_t.transpose(2, 0, 1)


# speedup vs baseline: 1.7489x; 1.7489x over previous
import numpy as np
import jax
import jax.numpy as jnp
from jax.experimental import pallas as pl
from jax.experimental.pallas import tpu as pltpu
from jax.experimental.pallas import tpu_sc as plsc

_B = 512
_TB = 16384


def kernel(x, table):
    batch, hist = x.shape
    vocab, dim = table.shape
    xt = x.T

    xt_s = (
        xt.reshape(hist, batch // _TB, 2, _TB // 2)
        .transpose(0, 1, 3, 2)
        .reshape(hist, batch)
    )

    mesh = plsc.VectorSubcoreMesh(core_axis_name="c", subcore_axis_name="s")

    @pl.kernel(
        out_type=jax.ShapeDtypeStruct((hist, batch, dim), table.dtype),
        mesh=mesh,
        compiler_params=pltpu.CompilerParams(use_tc_tiling_on_sc=False),
    )
    def gather_kernel(table_hbm, i_hbm, o_hbm):
        def body(i_vmem, o_vmem):
            pltpu.sync_copy(table_hbm.at[i_vmem.at[0]], o_vmem.at[0])

        pltpu.emit_pipeline(
            body,
            grid=(hist, batch // _B),
            in_specs=[pl.BlockSpec((1, _B), index_map=lambda h, b: (h, b))],
            out_specs=[
                pl.BlockSpec((1, _B, dim), index_map=lambda h, b: (h, b, 0))
            ],
            core_axis_name=("c", "s"),
            dimension_semantics=(pltpu.PARALLEL, pltpu.PARALLEL),
        )(i_hbm, o_hbm)

    g = gather_kernel(table, xt_s)

    g128 = g.reshape(hist, batch // 2, 2 * dim)

    def transpose_body(in_ref, out_ref):
        vt = in_ref[0].T
        out_ref[0] = jnp.concatenate([vt[:dim], vt[dim:]], axis=1)

    out_t = pl.pallas_call(
        transpose_body,
        grid=(hist, batch // _TB),
        in_specs=[
            pl.BlockSpec(
                (1, _TB // 2, 2 * dim), index_map=lambda h, b: (h, b, 0)
            )
        ],
        out_specs=pl.BlockSpec((1, dim, _TB), index_map=lambda h, b: (h, 0, b)),
        out_shape=jax.ShapeDtypeStruct((hist, dim, batch), table.dtype),
    )(g128)

    return out_t.transpose(2, 0, 1)
